# Initial kernel scaffold; baseline (speedup 1.0000x reference)
import functools

import jax
import jax.numpy as jnp
from jax import lax
from jax.experimental import pallas as pl
from jax.experimental.pallas import tpu as pltpu
from jax.experimental.pallas import tpu_sc as plsc

_R = 32
_V = _R ** 3
_B, _C, _N = 8, 64, 65536
_NC, _NS = 2, 16
_CH = _C // _NC
_PTS = _N // _NS
_CHUNK = 1024
_NCHUNK = _PTS // _CHUNK
_IDXROWS = _PTS // 128
_ROWS_PER_TEC = _V // _NS
_ZROWS = 256
_HR = _V // 128


def _norm_body(coords_ref, nc_ref, idx_ref):
    c = coords_ref[0]
    m = jnp.mean(c, axis=1, keepdims=True)
    cc = c - m
    nrm = jnp.sqrt(jnp.sum(cc * cc, axis=0, keepdims=True))
    denom = jnp.max(nrm) * 2.0
    ncd = cc / denom + 0.5
    ncd = jnp.clip(ncd * _R, 0.0, _R - 1.0)
    nc_ref[0] = ncd
    vox = jnp.round(ncd).astype(jnp.int32)
    idx_ref[0] = vox[0:1] * (_R * _R) + vox[1:2] * _R + vox[2:3]


def _norm_pallas(coords):
    return pl.pallas_call(
        _norm_body,
        grid=(_B,),
        in_specs=[pl.BlockSpec((1, 3, _N), lambda b: (b, 0, 0))],
        out_specs=[pl.BlockSpec((1, 3, _N), lambda b: (b, 0, 0)),
                   pl.BlockSpec((1, 1, _N), lambda b: (b, 0, 0))],
        out_shape=[jax.ShapeDtypeStruct((_B, 3, _N), jnp.float32),
                   jax.ShapeDtypeStruct((_B, 1, _N), jnp.int32)],
    )(coords)


_TB = 8192


def _trans_body(f_ref, o_ref):
    o_ref[0, 0] = f_ref[0].T


def _trans_pallas(features):
    return pl.pallas_call(
        _trans_body,
        grid=(_NC, _B, _N // _TB),
        in_specs=[pl.BlockSpec((1, _CH, _TB), lambda c, b, k: (b, c, k))],
        out_specs=[pl.BlockSpec((1, 1, _TB, _CH), lambda c, b, k: (c, b, k, 0))],
        out_shape=[jax.ShapeDtypeStruct((_NC, _B, _N, _CH), jnp.float32)],
    )(features)[0]


def _sc_body(trans_hbm, idx_hbm, sums_hbm, hist_hbm,
             table, fbuf, ibuf, hist, zbuf, sem):
    cid = lax.axis_index("c")
    sid = lax.axis_index("s")
    zero16 = jnp.zeros((16,), jnp.float32)

    @pl.loop(0, _ZROWS)
    def _(r):
        zbuf[r, pl.ds(0, 16)] = zero16
        zbuf[r, pl.ds(16, 16)] = zero16

    @pl.loop(0, _B)
    def _(b):
        row0 = sid * _ROWS_PER_TEC
        for k in range(_ROWS_PER_TEC // _ZROWS):
            pltpu.sync_copy(zbuf, table.at[pl.ds(row0 + k * _ZROWS, _ZROWS)])

        my_hist = lax.rem(b, _NC) == cid

        @pl.when(my_hist)
        def _():
            @pl.loop(0, _HR)
            def _(r):
                for s in range(8):
                    hist[r, pl.ds(s * 16, 16)] = zero16

        pltpu.sync_copy(idx_hbm.at[b, sid], ibuf)
        plsc.subcore_barrier()

        pbase = sid * _PTS

        @pl.loop(0, _NCHUNK)
        def _(ck):
            pltpu.sync_copy(
                trans_hbm.at[cid, b, pl.ds(pbase + ck * _CHUNK, _CHUNK)], fbuf)
            for j in range(_CHUNK // 128):
                pltpu.sync_copy(fbuf.at[pl.ds(j * 128, 128)],
                                table.at[ibuf.at[ck * (_CHUNK // 128) + j]],
                                add=True)

        @pl.when(my_hist)
        def _():
            one16 = jnp.full((16,), 1.0, jnp.float32)

            @pl.loop(0, _IDXROWS)
            def _(r):
                for s in range(8):
                    v = ibuf[r, pl.ds(s * 16, 16)]
                    plsc.addupdate_scatter(
                        hist,
                        [lax.shift_right_logical(v, 7),
                         lax.bitwise_and(v, 127)],
                        one16)

        plsc.subcore_barrier()

        for k in range(_ROWS_PER_TEC // _ZROWS):
            pltpu.sync_copy(table.at[pl.ds(row0 + k * _ZROWS, _ZROWS)],
                            sums_hbm.at[cid, b, pl.ds(row0 + k * _ZROWS, _ZROWS)])

        @pl.when(my_hist)
        def _():
            pltpu.sync_copy(hist, hist_hbm.at[b, sid])


def _sc_scatter(trans, idx4):
    mesh = plsc.VectorSubcoreMesh(core_axis_name="c", subcore_axis_name="s")
    return pl.kernel(
        _sc_body,
        out_type=(jax.ShapeDtypeStruct((_NC, _B, _V, _CH), jnp.float32),
                  jax.ShapeDtypeStruct((_B, _NS, _HR, 128), jnp.float32)),
        mesh=mesh,
        scratch_types=[
            pltpu.VMEM_SHARED((_V, _CH), jnp.float32),
            pltpu.VMEM((_CHUNK, _CH), jnp.float32),
            pltpu.VMEM((_IDXROWS, 128), jnp.int32),
            pltpu.VMEM((_HR, 128), jnp.float32),
            pltpu.VMEM((_ZROWS, _CH), jnp.float32),
            pltpu.SemaphoreType.DMA,
        ],
    )(trans, idx4)


_VB = 8192


def _final_body(s_ref, h_ref, o_ref):
    cnt = jnp.sum(h_ref[0], axis=0)
    cnt = jnp.maximum(cnt, 1.0)
    o_ref[0] = s_ref[0, 0].T / cnt[None, :]


def _final_pallas(sums, hists_blk):
    return pl.pallas_call(
        _final_body,
        grid=(_B, _NC, _V // _VB),
        in_specs=[
            pl.BlockSpec((1, 1, _VB, _CH), lambda b, c, k: (c, b, k, 0)),
            pl.BlockSpec((1, _NS, _VB), lambda b, c, k: (b, 0, k)),
        ],
        out_specs=[pl.BlockSpec((1, _CH, _VB), lambda b, c, k: (b, c, k))],
        out_shape=[jax.ShapeDtypeStruct((_B, _C, _V), jnp.float32)],
    )(sums, hists_blk)[0]


def kernel(features, coords):
    norm_coords, idx = _norm_pallas(coords)
    norm_coords = norm_coords.reshape(_B, 3, _N)
    idx4 = idx.reshape(_B, _NS, _IDXROWS, 128)
    trans = _trans_pallas(features)
    sums, hists = _sc_scatter(trans, idx4)
    hists_r = hists.reshape(_B, _NS, _V)
    out = _final_pallas(sums, hists_r)
    return out.reshape(_B, _C, _R, _R, _R), norm_coords


# speedup vs baseline: 1.7132x; 1.7132x over previous
import dataclasses
import functools

import jax
import jax.numpy as jnp
from jax import lax
from jax.experimental import pallas as pl
from jax.experimental.pallas import tpu as pltpu
from jax.experimental.pallas import tpu_sc as plsc

_R = 32
_V = _R ** 3
_B, _C, _N = 8, 64, 65536
_NC, _NS = 2, 16
_CH = _C // _NC
_PTS = _N // _NS
_CHUNK = 512
_NCHUNK = _PTS // _CHUNK
_IDXROWS = _PTS // 128
_ROWS_PER_TEC = _V // _NS
_ZROWS = 256
_HR = _V // 128


def _norm_body(coords_ref, nc_ref, idx_ref):
    c = coords_ref[0]
    m = jnp.mean(c, axis=1, keepdims=True)
    cc = c - m
    nrm = jnp.sqrt(jnp.sum(cc * cc, axis=0, keepdims=True))
    denom = jnp.max(nrm) * 2.0
    ncd = cc / denom + 0.5
    ncd = jnp.clip(ncd * _R, 0.0, _R - 1.0)
    nc_ref[0] = ncd
    vox = jnp.round(ncd).astype(jnp.int32)
    idx_ref[0] = vox[0:1] * (_R * _R) + vox[1:2] * _R + vox[2:3]


def _norm_pallas(coords):
    return pl.pallas_call(
        _norm_body,
        grid=(_B,),
        in_specs=[pl.BlockSpec((1, 3, _N), lambda b: (b, 0, 0))],
        out_specs=[pl.BlockSpec((1, 3, _N), lambda b: (b, 0, 0)),
                   pl.BlockSpec((1, 1, _N), lambda b: (b, 0, 0))],
        out_shape=[jax.ShapeDtypeStruct((_B, 3, _N), jnp.float32),
                   jax.ShapeDtypeStruct((_B, 1, _N), jnp.int32)],
    )(coords)


_TB = 8192


def _trans_body(f_ref, o_ref):
    o_ref[0, 0] = f_ref[0].T


def _trans_pallas(features):
    return pl.pallas_call(
        _trans_body,
        grid=(_NC, _B, _N // _TB),
        in_specs=[pl.BlockSpec((1, _CH, _TB), lambda c, b, k: (b, c, k))],
        out_specs=[pl.BlockSpec((1, 1, _TB, _CH), lambda c, b, k: (c, b, k, 0))],
        out_shape=[jax.ShapeDtypeStruct((_NC, _B, _N, _CH), jnp.float32)],
    )(features)[0]


def _sc_body(trans_hbm, idx_hbm, sums_hbm, hist_hbm,
             table, fbuf, ibuf, hist, zbuf, sem):
    cid = lax.axis_index("c")
    sid = lax.axis_index("s")
    zero16 = jnp.zeros((16,), jnp.float32)

    @pl.loop(0, _ZROWS)
    def _(r):
        zbuf[r, pl.ds(0, 16)] = zero16
        zbuf[r, pl.ds(16, 16)] = zero16

    @pl.loop(0, _B)
    def _(b):
        row0 = sid * _ROWS_PER_TEC
        for k in range(_ROWS_PER_TEC // _ZROWS):
            pltpu.sync_copy(zbuf, table.at[pl.ds(row0 + k * _ZROWS, _ZROWS)])

        my_hist = lax.rem(b, _NC) == cid

        @pl.when(my_hist)
        def _():
            @pl.loop(0, _HR)
            def _(r):
                for s in range(8):
                    hist[r, pl.ds(s * 16, 16)] = zero16

        pltpu.sync_copy(idx_hbm.at[b, sid], ibuf)
        plsc.subcore_barrier()

        pbase = sid * _PTS

        @pl.loop(0, _NCHUNK)
        def _(ck):
            pltpu.sync_copy(
                trans_hbm.at[cid, b, pl.ds(pbase + ck * _CHUNK, _CHUNK)], fbuf)
            for j in range(_CHUNK // 128):
                pltpu.sync_copy(fbuf.at[pl.ds(j * 128, 128)],
                                table.at[ibuf.at[ck * (_CHUNK // 128) + j]],
                                add=True)

        @pl.when(my_hist)
        def _():
            one16 = jnp.full((16,), 1.0, jnp.float32)

            @pl.loop(0, _IDXROWS)
            def _(r):
                for s in range(8):
                    v = ibuf[r, pl.ds(s * 16, 16)]
                    plsc.addupdate_scatter(
                        hist,
                        [lax.shift_right_logical(v, 7),
                         lax.bitwise_and(v, 127)],
                        one16)

        plsc.subcore_barrier()

        for k in range(_ROWS_PER_TEC // _ZROWS):
            pltpu.sync_copy(table.at[pl.ds(row0 + k * _ZROWS, _ZROWS)],
                            sums_hbm.at[cid, b, pl.ds(row0 + k * _ZROWS, _ZROWS)])

        @pl.when(my_hist)
        def _():
            pltpu.sync_copy(hist, hist_hbm.at[b, sid])


def _sc_compiler_params():
    cp = pltpu.CompilerParams(use_tc_tiling_on_sc=False)
    if "needs_layout_passes" in pltpu.CompilerParams.__dataclass_fields__:
        cp = dataclasses.replace(cp, needs_layout_passes=False)
    return cp


def _sc_scatter(trans, idx4):
    mesh = plsc.VectorSubcoreMesh(core_axis_name="c", subcore_axis_name="s")
    return pl.kernel(
        _sc_body,
        compiler_params=_sc_compiler_params(),
        out_type=(jax.ShapeDtypeStruct((_NC, _B, _V, _CH), jnp.float32),
                  jax.ShapeDtypeStruct((_B, _NS, _HR, 128), jnp.float32)),
        mesh=mesh,
        scratch_types=[
            pltpu.VMEM_SHARED((_V, _CH), jnp.float32),
            pltpu.VMEM((_CHUNK, _CH), jnp.float32),
            pltpu.VMEM((_IDXROWS, 128), jnp.int32),
            pltpu.VMEM((_HR, 128), jnp.float32),
            pltpu.VMEM((_ZROWS, _CH), jnp.float32),
            pltpu.SemaphoreType.DMA,
        ],
    )(trans, idx4)


_VB = 8192


def _final_body(s_ref, h_ref, o_ref):
    cnt = jnp.sum(h_ref[0], axis=0)
    cnt = jnp.maximum(cnt, 1.0)
    o_ref[0] = s_ref[0, 0].T / cnt[None, :]


def _final_pallas(sums, hists_blk):
    return pl.pallas_call(
        _final_body,
        grid=(_B, _NC, _V // _VB),
        in_specs=[
            pl.BlockSpec((1, 1, _VB, _CH), lambda b, c, k: (c, b, k, 0)),
            pl.BlockSpec((1, _NS, _VB), lambda b, c, k: (b, 0, k)),
        ],
        out_specs=[pl.BlockSpec((1, _CH, _VB), lambda b, c, k: (b, c, k))],
        out_shape=[jax.ShapeDtypeStruct((_B, _C, _V), jnp.float32)],
    )(sums, hists_blk)[0]


def kernel(features, coords):
    norm_coords, idx = _norm_pallas(coords)
    norm_coords = norm_coords.reshape(_B, 3, _N)
    idx4 = idx.reshape(_B, _NS, _IDXROWS, 128)
    trans = _trans_pallas(features)
    sums, hists = _sc_scatter(trans, idx4)
    hists_r = hists.reshape(_B, _NS, _V)
    out = _final_pallas(sums, hists_r)
    return out.reshape(_B, _C, _R, _R, _R), norm_coords
